# baseline (speedup 1.0000x reference)
import functools

import jax
import jax.numpy as jnp
from jax import lax
from jax.experimental import pallas as pl
from jax.experimental.pallas import tpu as pltpu
from jax.experimental.pallas import tpu_sc as plsc

_C = 64
_QH = 32
_KH = 48
_P = _KH - _QH + 1
_N = _C * _QH * _QH
_LW = _QH * _C
_KW = _KH * _C
_NY = _P * _P
_NPAD = 304


def _patch_kernel(q_ref, ka_ref, kb_ref, y_ref):

    def make_a_body(r):
        def a_body(a, _):
            di = a * 8 + r
            accs = [jnp.zeros((8, 128), jnp.float32) for _ in range(_P)]
            for rb in range(4):
                base = pl.multiple_of((a + rb) * 8, 8)
                nrows = 8 if r == 0 else 16
                qb = q_ref[rb * 8:(rb + 1) * 8, :]
                sa = ka_ref[pl.ds(base, nrows), :]
                sb = kb_ref[pl.ds(base, nrows), :]
                if r:
                    sa = jax.lax.slice(sa, (r, 0), (r + 8, _KW))
                    sb = jax.lax.slice(sb, (r, 0), (r + 8, _KW))
                for dj in range(_P):
                    src = sb if (dj % 2) else sa
                    off = (dj // 2) * 128
                    w = jax.lax.slice(src, (0, off), (8, off + _LW))
                    d = jnp.abs(w - qb)
                    for c in range(_LW // 128):
                        accs[dj] = accs[dj] + jax.lax.slice(
                            d, (0, 128 * c), (8, 128 * (c + 1)))
            for dj in range(_P):
                y_ref[di, dj] = jnp.sum(accs[dj])
            return 0
        return a_body

    for r in range(8):
        n_a = 3 if r == 0 else 2
        jax.lax.fori_loop(0, n_a, make_a_body(r), 0)


def _sc_argmin(y_hbm, out_hbm, yv, ov):
    wid = lax.axis_index("s") * 2 + lax.axis_index("c")

    @pl.when(wid == 0)
    def _():
        pltpu.sync_copy(y_hbm, yv)
        lanes = lax.iota(jnp.int32, 16)
        minvec = yv[pl.ds(0, 16)]
        idxvec = jnp.zeros((16,), jnp.int32)
        for i in range(1, _NPAD // 16):
            v = yv[pl.ds(i * 16, 16)]
            lt = v < minvec
            minvec = jnp.where(lt, v, minvec)
            idxvec = jnp.where(lt, jnp.full((16,), i, jnp.int32), idxvec)
        m = jnp.min(minvec)
        flat = idxvec * 16 + lanes
        cand = jnp.where(minvec == m, flat, jnp.int32(2**30))
        bi = jnp.min(cand)
        ovec = jnp.where(
            lanes == 0, bi.astype(jnp.float32),
            jnp.where(lanes == 1, m * jnp.float32(1.0 / _N), jnp.float32(0)))
        ov[...] = ovec
        pltpu.sync_copy(ov, out_hbm)


def kernel(query, key):
    P = int(key.shape[3]) - int(query.shape[3]) + 1

    q = query[0].transpose(1, 2, 0).reshape(_QH, _LW)
    k3 = key[0].transpose(1, 2, 0)
    ka = k3.reshape(_KH, _KW)
    kb = jnp.pad(k3[:, 1:, :], ((0, 0), (0, 1), (0, 0))).reshape(_KH, _KW)

    y = pl.pallas_call(
        _patch_kernel,
        out_shape=jax.ShapeDtypeStruct((_P, _P), jnp.float32),
        out_specs=pl.BlockSpec(memory_space=pltpu.SMEM),
    )(q, ka, kb)

    ypad = jnp.concatenate(
        [y.reshape(_NY), jnp.full((_NPAD - _NY,), jnp.inf, jnp.float32)])

    mesh = plsc.VectorSubcoreMesh(core_axis_name="c", subcore_axis_name="s")
    sc = functools.partial(
        pl.kernel,
        mesh=mesh,
        out_type=jax.ShapeDtypeStruct((16,), jnp.float32),
        scratch_types=[
            pltpu.VMEM((_NPAD,), jnp.float32),
            pltpu.VMEM((16,), jnp.float32),
        ],
        compiler_params=pltpu.CompilerParams(needs_layout_passes=False),
    )(_sc_argmin)
    out = sc(ypad)

    hard = out[0].astype(jnp.int32).reshape(1)
    rel = out[1].reshape(1, 1)
    return (hard, P, rel)


# speedup vs baseline: 1.7951x; 1.7951x over previous
import jax
import jax.numpy as jnp
from jax.experimental import pallas as pl
from jax.experimental.pallas import tpu as pltpu

_C = 64
_QH = 32
_KH = 48
_P = _KH - _QH + 1
_N = _C * _QH * _QH
_LW = _QH * _C
_KW = _KH * _C


def _patch_kernel(q_ref, ka_ref, kb_ref, idx_ref, val_ref):

    def make_a_body(r):
        def a_body(a, carry):
            best_val, best_idx = carry
            di = a * 8 + r
            accs = [jnp.zeros((8, 128), jnp.float32) for _ in range(_P)]
            for rb in range(4):
                base = pl.multiple_of((a + rb) * 8, 8)
                nrows = 8 if r == 0 else 16
                qb = q_ref[rb * 8:(rb + 1) * 8, :]
                sa = ka_ref[pl.ds(base, nrows), :]
                sb = kb_ref[pl.ds(base, nrows), :]
                if r:
                    sa = jax.lax.slice(sa, (r, 0), (r + 8, _KW))
                    sb = jax.lax.slice(sb, (r, 0), (r + 8, _KW))
                for dj in range(_P):
                    src = sb if (dj % 2) else sa
                    off = (dj // 2) * 128
                    w = jax.lax.slice(src, (0, off), (8, off + _LW))
                    d = jnp.abs(w - qb)
                    for c in range(_LW // 128):
                        accs[dj] = accs[dj] + jax.lax.slice(
                            d, (0, 128 * c), (8, 128 * (c + 1)))
            for dj in range(_P):
                s = jnp.sum(accs[dj])
                idx = di * _P + dj
                take = (s < best_val) | ((s == best_val) & (idx < best_idx))
                best_val = jnp.where(take, s, best_val)
                best_idx = jnp.where(take, idx, best_idx)
            return best_val, best_idx
        return a_body

    carry = (jnp.float32(jnp.inf), jnp.int32(2**30))
    for r in range(8):
        n_a = 3 if r == 0 else 2
        carry = jax.lax.fori_loop(0, n_a, make_a_body(r), carry)
    best_val, best_idx = carry
    idx_ref[0] = best_idx
    val_ref[0, 0] = best_val / jnp.float32(_N)


def kernel(query, key):
    P = int(key.shape[3]) - int(query.shape[3]) + 1

    q = query[0].transpose(1, 2, 0).reshape(_QH, _LW)
    k3 = key[0].transpose(1, 2, 0)
    ka = k3.reshape(_KH, _KW)
    kb = jnp.pad(k3[:, 1:, :], ((0, 0), (0, 1), (0, 0))).reshape(_KH, _KW)

    idx, val = pl.pallas_call(
        _patch_kernel,
        out_shape=(
            jax.ShapeDtypeStruct((1,), jnp.int32),
            jax.ShapeDtypeStruct((1, 1), jnp.float32),
        ),
        out_specs=(
            pl.BlockSpec(memory_space=pltpu.SMEM),
            pl.BlockSpec(memory_space=pltpu.SMEM),
        ),
    )(q, ka, kb)

    return (idx, P, val)
